# initial kernel scaffold (unmeasured)
import jax
import jax.numpy as jnp
from jax import lax
from jax.experimental import pallas as pl
from jax.experimental.pallas import tpu as pltpu


def kernel(
    x,
):
    def body(*refs):
        pass

    out_shape = jax.ShapeDtypeStruct(..., jnp.float32)
    return pl.pallas_call(body, out_shape=out_shape)(...)


# baseline (device time: 18635 ns/iter reference)
import jax
import jax.numpy as jnp
from jax import lax
from jax.experimental import pallas as pl
from jax.experimental.pallas import tpu as pltpu


def kernel(x):
    m, n = x.shape
    out_cols = n // 2

    def body(x_ref, out_ref, send_buf, recv_buf, send_sem, recv_sem):
        my_x = lax.axis_index("x")
        my_y = lax.axis_index("y")
        other_x = 1 - my_x

        barrier_sem = pltpu.get_barrier_semaphore()
        pl.semaphore_signal(
            barrier_sem,
            inc=1,
            device_id=(other_x, my_y),
            device_id_type=pl.DeviceIdType.MESH,
        )
        pl.semaphore_wait(barrier_sem, 1)

        out_ref[pl.ds(my_x * m, m), :] = x_ref[
            :, pl.ds(my_x * out_cols, out_cols)
        ].astype(jnp.bfloat16)

        send_buf[...] = x_ref[:, pl.ds(other_x * out_cols, out_cols)].astype(
            jnp.bfloat16
        )
        rdma = pltpu.make_async_remote_copy(
            src_ref=send_buf,
            dst_ref=recv_buf,
            send_sem=send_sem,
            recv_sem=recv_sem,
            device_id=(other_x, my_y),
            device_id_type=pl.DeviceIdType.MESH,
        )
        rdma.start()
        rdma.wait()

        out_ref[pl.ds(other_x * m, m), :] = recv_buf[...]

    out_shape = jax.ShapeDtypeStruct((2 * m, out_cols), jnp.bfloat16)
    return pl.pallas_call(
        body,
        out_shape=out_shape,
        in_specs=[pl.BlockSpec(memory_space=pltpu.VMEM)],
        out_specs=pl.BlockSpec(memory_space=pltpu.VMEM),
        scratch_shapes=[
            pltpu.VMEM((m, out_cols), jnp.bfloat16),
            pltpu.VMEM((m, out_cols), jnp.bfloat16),
            pltpu.SemaphoreType.DMA,
            pltpu.SemaphoreType.DMA,
        ],
        compiler_params=pltpu.CompilerParams(collective_id=0),
    )(x)


# device time: 16170 ns/iter; 1.1524x vs baseline; 1.1524x over previous
import jax
import jax.numpy as jnp
from jax import lax
from jax.experimental import pallas as pl
from jax.experimental.pallas import tpu as pltpu

NB = 8


def kernel(x):
    m, n = x.shape
    out_cols = n // 2
    half = m // 2
    br = half // NB

    def body(x_ref, out_ref, send_stage, x_send_sems, x_recv_sems,
             y_send_sems, y_recv_sems):
        my_x = lax.axis_index("x")
        my_y = lax.axis_index("y")
        other_x = 1 - my_x
        other_y = 1 - my_y

        barrier_sem = pltpu.get_barrier_semaphore()
        for dev in [(other_x, my_y), (my_x, other_y)]:
            pl.semaphore_signal(
                barrier_sem, inc=1, device_id=dev,
                device_id_type=pl.DeviceIdType.MESH,
            )
        pl.semaphore_wait(barrier_sem, 2)

        dst_base = my_x * m + my_y * half
        recv_base = other_x * m + my_y * half

        x_rdmas = []
        for b in range(NB):
            rows = pl.ds(my_y * half + b * br, br)
            send_stage[pl.ds(b * br, br), :] = x_ref[
                rows, pl.ds(other_x * out_cols, out_cols)
            ].astype(jnp.bfloat16)
            rdma = pltpu.make_async_remote_copy(
                src_ref=send_stage.at[pl.ds(b * br, br)],
                dst_ref=out_ref.at[pl.ds(dst_base + b * br, br)],
                send_sem=x_send_sems.at[b],
                recv_sem=x_recv_sems.at[b],
                device_id=(other_x, my_y),
                device_id_type=pl.DeviceIdType.MESH,
            )
            rdma.start()
            x_rdmas.append(rdma)

        out_ref[pl.ds(my_x * m, m), :] = x_ref[
            :, pl.ds(my_x * out_cols, out_cols)
        ].astype(jnp.bfloat16)

        y_rdmas = []
        for b in range(NB):
            x_rdmas[b].wait_recv()
            rdma = pltpu.make_async_remote_copy(
                src_ref=out_ref.at[pl.ds(recv_base + b * br, br)],
                dst_ref=out_ref.at[pl.ds(recv_base + b * br, br)],
                send_sem=y_send_sems.at[b],
                recv_sem=y_recv_sems.at[b],
                device_id=(my_x, other_y),
                device_id_type=pl.DeviceIdType.MESH,
            )
            rdma.start()
            y_rdmas.append(rdma)

        for b in range(NB):
            y_rdmas[b].wait_recv()
        for b in range(NB):
            x_rdmas[b].wait_send()
            y_rdmas[b].wait_send()

    out_shape = jax.ShapeDtypeStruct((2 * m, out_cols), jnp.bfloat16)
    return pl.pallas_call(
        body,
        out_shape=out_shape,
        in_specs=[pl.BlockSpec(memory_space=pltpu.VMEM)],
        out_specs=pl.BlockSpec(memory_space=pltpu.VMEM),
        scratch_shapes=[
            pltpu.VMEM((half, out_cols), jnp.bfloat16),
            pltpu.SemaphoreType.DMA((NB,)),
            pltpu.SemaphoreType.DMA((NB,)),
            pltpu.SemaphoreType.DMA((NB,)),
            pltpu.SemaphoreType.DMA((NB,)),
        ],
        compiler_params=pltpu.CompilerParams(collective_id=0),
    )(x)
